# baseline (speedup 1.0000x reference)
import functools

import jax
import jax.numpy as jnp
from jax import lax
from jax.experimental import pallas as pl
from jax.experimental.pallas import tpu as pltpu
from jax.experimental.pallas import tpu_sc as plsc

E = 64
D = 768
RH = 768
HID = 3072
NT = 2048
NW = 32
TPW = NT // NW
BLK = 64
EPG = 2
NPAD = 2560
_NC = 2


def _router_body(x_ref, wr_ref, bias_ref, score_ref, scoreb_ref, pos_ref,
                 offs_ref, cnts_ref):
    x = x_ref[...]
    wr = wr_ref[...]
    logits = lax.dot_general(x, wr, (((1,), (1,)), ((), ())),
                             preferred_element_type=jnp.float32)
    scores = jax.nn.sigmoid(logits)
    sel = scores + bias_ref[...]
    m = jnp.max(sel, axis=1, keepdims=True)
    iota = lax.broadcasted_iota(jnp.int32, (NT, E), 1)
    idx = jnp.min(jnp.where(sel == m, iota, E), axis=1, keepdims=True)
    score = jnp.sum(jnp.where(iota == idx, scores, 0.0), axis=1,
                    keepdims=True)
    score_ref[...] = score
    scoreb_ref[...] = jnp.broadcast_to(score, (NT, 16))
    oh = (iota == idx).astype(jnp.float32)
    tot = jnp.sum(oh, axis=0, keepdims=True)
    toti = tot.astype(jnp.int32)
    padi = (toti + 7) & (-8)
    cnts_ref[...] = toti
    ei = lax.broadcasted_iota(jnp.int32, (E, E), 0)
    ej = lax.broadcasted_iota(jnp.int32, (E, E), 1)
    texc = (ei < ej).astype(jnp.float32)
    offs_f = lax.dot_general(padi.astype(jnp.float32), texc,
                             (((1,), (0,)), ((), ())),
                             preferred_element_type=jnp.float32,
                             precision=lax.Precision.HIGHEST)
    offs_ref[...] = offs_f.astype(jnp.int32)
    ti = lax.broadcasted_iota(jnp.int32, (TPW, TPW), 0)
    tj = lax.broadcasted_iota(jnp.int32, (TPW, TPW), 1)
    tlow = (tj < ti).astype(jnp.float32)
    prior = offs_f
    for w in range(NW):
        ohw = oh[w * TPW:(w + 1) * TPW, :]
        ranks = lax.dot_general(tlow, ohw, (((1,), (0,)), ((), ())),
                                preferred_element_type=jnp.float32,
                                precision=lax.Precision.HIGHEST)
        posw = jnp.sum((ranks + prior) * ohw, axis=1, keepdims=True)
        pos_ref[w * TPW:(w + 1) * TPW, :] = posw.astype(jnp.int32)
        prior = prior + jnp.sum(ohw, axis=0, keepdims=True)


def _router(x2d, w_router, expert_bias):
    return pl.pallas_call(
        _router_body,
        out_shape=(
            jax.ShapeDtypeStruct((NT, 1), jnp.float32),
            jax.ShapeDtypeStruct((NT, 16), jnp.float32),
            jax.ShapeDtypeStruct((NT, 1), jnp.int32),
            jax.ShapeDtypeStruct((1, E), jnp.int32),
            jax.ShapeDtypeStruct((1, E), jnp.int32),
        ),
    )(x2d, w_router, expert_bias.reshape(1, E))


def _dispatch_body(pos_hbm, x_hbm, xs_hbm, pos_v, rows_v, sem):
    c = lax.axis_index("c")
    s = lax.axis_index("s")
    wid = s * _NC + c
    tbase = wid * TPW
    pltpu.sync_copy(pos_hbm.at[pl.ds(tbase, TPW)], pos_v)
    pltpu.sync_copy(x_hbm.at[pl.ds(tbase, TPW)], rows_v)
    pltpu.async_copy(rows_v, xs_hbm.at[pos_v], sem).wait()


def _dispatch(pos_flat, x2d):
    mesh = plsc.VectorSubcoreMesh(core_axis_name="c", subcore_axis_name="s")
    fn = pl.kernel(
        _dispatch_body,
        out_type=jax.ShapeDtypeStruct((NPAD, D), jnp.float32),
        mesh=mesh,
        scratch_types=[
            pltpu.VMEM((TPW,), jnp.int32),
            pltpu.VMEM((TPW, D), jnp.float32),
            pltpu.SemaphoreType.DMA,
        ],
    )
    return fn(pos_flat, x2d)


HC = HID // 8
SH_TOK = NT // 2


def _egrid(s):
    return s - (s + 1) // 5


def _fused_body(offs_ref, cnts_ref, xs_ref, x_ref, w1_ref, w2_ref, w3_ref,
                ws1_ref, ws2_ref, ws3_ref, out_ref, sh_ref):
    s = pl.program_id(0)
    is_shared = lax.rem(s, 5) == 4

    @pl.when(jnp.logical_not(is_shared))
    def _expert():
        e = _egrid(s)
        off = pl.multiple_of(offs_ref[e], 8)
        cnt = cnts_ref[e]
        nblk = lax.div(cnt + (BLK - 1), BLK)
        w1 = w1_ref[...]
        w2 = w2_ref[...]
        w3 = w3_ref[...]

        def blk(i, carry):
            base = off + i * BLK
            rows = xs_ref[pl.ds(base, BLK), :]
            h1 = lax.dot_general(rows, w1, (((1,), (1,)), ((), ())),
                                 preferred_element_type=jnp.float32)
            h2 = lax.dot_general(rows, w2, (((1,), (1,)), ((), ())),
                                 preferred_element_type=jnp.float32)
            h = h1 * jax.nn.sigmoid(h1) * h2
            y = lax.dot_general(h, w3, (((1,), (1,)), ((), ())),
                                preferred_element_type=jnp.float32)
            out_ref[pl.ds(base, BLK), :] = y
            return carry

        lax.fori_loop(0, nblk, blk, 0)

    @pl.when(is_shared)
    def _shared_step():
        m = lax.div(s, 5)
        half = lax.rem(m, 2)
        c = lax.div(m, 2)
        rows = pl.ds(half * SH_TOK, SH_TOK)
        xh = x_ref[rows, :]
        h1 = lax.dot_general(xh, ws1_ref[...], (((1,), (1,)), ((), ())),
                             preferred_element_type=jnp.float32)
        h2 = lax.dot_general(xh, ws2_ref[...], (((1,), (1,)), ((), ())),
                             preferred_element_type=jnp.float32)
        h = h1 * jax.nn.sigmoid(h1) * h2
        part = lax.dot_general(h, ws3_ref[...], (((1,), (1,)), ((), ())),
                               preferred_element_type=jnp.float32)

        @pl.when(c == 0)
        def _init():
            sh_ref[rows, :] = part

        @pl.when(c != 0)
        def _acc():
            sh_ref[rows, :] = sh_ref[rows, :] + part


def _grouped_shared(offs, cnts, xs, w12, w3, x2d, w1s, w2s, w3s):
    return pl.pallas_call(
        _fused_body,
        grid=(80,),
        in_specs=[
            pl.BlockSpec(memory_space=pltpu.SMEM),
            pl.BlockSpec(memory_space=pltpu.SMEM),
            pl.BlockSpec((NPAD, D), lambda s: (0, 0)),
            pl.BlockSpec((NT, D), lambda s: (0, 0)),
            pl.BlockSpec((None, RH, D), lambda s: (_egrid(s), 0, 0)),
            pl.BlockSpec((None, RH, D), lambda s: (_egrid(s), 1, 0)),
            pl.BlockSpec((None, D, RH), lambda s: (_egrid(s), 0, 0)),
            pl.BlockSpec((HC, D), lambda s: (s // 10, 0)),
            pl.BlockSpec((HC, D), lambda s: (s // 10, 0)),
            pl.BlockSpec((D, HC), lambda s: (0, s // 10)),
        ],
        out_specs=(
            pl.BlockSpec((NPAD, D), lambda s: (0, 0)),
            pl.BlockSpec((NT, D), lambda s: (0, 0)),
        ),
        out_shape=(
            jax.ShapeDtypeStruct((NPAD, D), jnp.float32),
            jax.ShapeDtypeStruct((NT, D), jnp.float32),
        ),
    )(offs, cnts, xs, x2d, w12, w12, w3, w1s, w2s, w3s)


def _ungather_body(outs_hbm, pos_hbm, sh_hbm, scb_hbm, final_hbm,
                   pos_v, rows_v, sh_v, sc_v, sem):
    c = lax.axis_index("c")
    s = lax.axis_index("s")
    wid = s * _NC + c
    tbase = wid * TPW
    pltpu.sync_copy(pos_hbm.at[pl.ds(tbase, TPW)], pos_v)
    gather = pltpu.async_copy(outs_hbm.at[pos_v], rows_v, sem)
    pltpu.sync_copy(sh_hbm.at[pl.ds(tbase, TPW)], sh_v)
    pltpu.sync_copy(scb_hbm.at[pl.ds(tbase, TPW)], sc_v)
    gather.wait()

    def tok(t, carry):
        sc16 = sc_v[t, :]
        for j in range(D // 16):
            col = pl.ds(16 * j, 16)
            rows_v[t, col] = sh_v[t, col] + sc16 * rows_v[t, col]
        return carry

    lax.fori_loop(0, TPW, tok, 0)
    pltpu.sync_copy(rows_v, final_hbm.at[pl.ds(tbase, TPW)])


def _ungather_final(out_sorted, pos, shared, scoreb):
    mesh = plsc.VectorSubcoreMesh(core_axis_name="c", subcore_axis_name="s")
    fn = pl.kernel(
        _ungather_body,
        out_type=jax.ShapeDtypeStruct((NT, D), jnp.float32),
        mesh=mesh,
        scratch_types=[
            pltpu.VMEM((TPW,), jnp.int32),
            pltpu.VMEM((TPW, D), jnp.float32),
            pltpu.VMEM((TPW, D), jnp.float32),
            pltpu.VMEM((TPW, 16), jnp.float32),
            pltpu.SemaphoreType.DMA,
        ],
    )
    return fn(out_sorted, pos, shared, scoreb)


def _shared_body(x_ref, w1_ref, w2_ref, w3s_ref, out_ref):
    xb = x_ref[...]
    h1 = lax.dot_general(xb, w1_ref[...], (((1,), (1,)), ((), ())),
                         preferred_element_type=jnp.float32)
    h2 = lax.dot_general(xb, w2_ref[...], (((1,), (1,)), ((), ())),
                         preferred_element_type=jnp.float32)
    h = h1 * jax.nn.sigmoid(h1) * h2
    out_ref[...] = lax.dot_general(h, w3s_ref[...], (((1,), (1,)), ((), ())),
                                   preferred_element_type=jnp.float32)


def _shared(x2d, w1s, w2s, w3s):
    TBLK = 512
    return pl.pallas_call(
        _shared_body,
        grid=(NT // TBLK,),
        in_specs=[
            pl.BlockSpec((TBLK, D), lambda t: (t, 0)),
            pl.BlockSpec((HID, D), lambda t: (0, 0)),
            pl.BlockSpec((HID, D), lambda t: (0, 0)),
            pl.BlockSpec((D, HID), lambda t: (0, 0)),
        ],
        out_specs=pl.BlockSpec((TBLK, D), lambda t: (t, 0)),
        out_shape=jax.ShapeDtypeStruct((NT, D), jnp.float32),
    )(x2d, w1s, w2s, w3s)


def _final_body(sh_ref, rt_ref, sc_ref, out_ref):
    out_ref[...] = sh_ref[...] + sc_ref[...] * rt_ref[...]


def _final(shared, routed, score):
    return pl.pallas_call(
        _final_body,
        out_shape=jax.ShapeDtypeStruct((NT, D), jnp.float32),
    )(shared, routed, score)


def kernel(x, w12, w3, w1s, w2s, w3s, w_router, expert_bias):
    b, s, d = x.shape
    x2d = x.reshape(NT, D)
    score2d, scoreb, pos2d, offs2d, cnts2d = _router(x2d, w_router,
                                                     expert_bias)
    xs = _dispatch(pos2d.reshape(NT), x2d)
    out_sorted, shared = _grouped_shared(offs2d.reshape(E), cnts2d.reshape(E),
                                         xs, w12, w3, x2d, w1s, w2s, w3s)
    out = _ungather_final(out_sorted, pos2d.reshape(NT), shared, scoreb)
    return out.reshape(b, s, d)


# speedup vs baseline: 1.1237x; 1.1237x over previous
import jax
import jax.numpy as jnp
from jax import lax
from jax.experimental import pallas as pl
from jax.experimental.pallas import tpu as pltpu
from jax.experimental.pallas import tpu_sc as plsc

E = 64
D = 768
RH = 768
HID = 3072
NT = 2048
NW = 32
TPW = NT // NW
BLK = 64
EPG = 2
NPAD = 2560
_NC = 2


def _router_body(x_ref, wr_ref, bias_ref, scoreb_ref, pos_ref,
                 offs_ref, cnts_ref):
    x = x_ref[...]
    wr = wr_ref[...]
    logits = lax.dot_general(x, wr, (((1,), (1,)), ((), ())),
                             preferred_element_type=jnp.float32)
    scores = jax.nn.sigmoid(logits)
    sel = scores + bias_ref[...]
    m = jnp.max(sel, axis=1, keepdims=True)
    iota = lax.broadcasted_iota(jnp.int32, (NT, E), 1)
    idx = jnp.min(jnp.where(sel == m, iota, E), axis=1, keepdims=True)
    score = jnp.sum(jnp.where(iota == idx, scores, 0.0), axis=1,
                    keepdims=True)
    scoreb_ref[...] = jnp.broadcast_to(score, (NT, 16))
    oh = (iota == idx).astype(jnp.float32)
    tot = jnp.sum(oh, axis=0, keepdims=True)
    toti = tot.astype(jnp.int32)
    padi = (toti + 7) & (-8)
    cnts_ref[...] = toti
    ei = lax.broadcasted_iota(jnp.int32, (E, E), 0)
    ej = lax.broadcasted_iota(jnp.int32, (E, E), 1)
    texc = (ei < ej).astype(jnp.float32)
    offs_f = lax.dot_general(padi.astype(jnp.float32), texc,
                             (((1,), (0,)), ((), ())),
                             preferred_element_type=jnp.float32,
                             precision=lax.Precision.HIGHEST)
    offs_ref[...] = offs_f.astype(jnp.int32)
    ti = lax.broadcasted_iota(jnp.int32, (TPW, TPW), 0)
    tj = lax.broadcasted_iota(jnp.int32, (TPW, TPW), 1)
    tlow = (tj < ti).astype(jnp.float32)
    prior = offs_f
    for w in range(NW):
        ohw = oh[w * TPW:(w + 1) * TPW, :]
        ranks = lax.dot_general(tlow, ohw, (((1,), (0,)), ((), ())),
                                preferred_element_type=jnp.float32,
                                precision=lax.Precision.HIGHEST)
        posw = jnp.sum((ranks + prior) * ohw, axis=1, keepdims=True)
        pos_ref[w * TPW:(w + 1) * TPW, :] = posw.astype(jnp.int32)
        prior = prior + jnp.sum(ohw, axis=0, keepdims=True)


def _router(x2d, w_router, expert_bias):
    return pl.pallas_call(
        _router_body,
        out_shape=(
            jax.ShapeDtypeStruct((NT, 16), jnp.float32),
            jax.ShapeDtypeStruct((NT, 1), jnp.int32),
            jax.ShapeDtypeStruct((1, E), jnp.int32),
            jax.ShapeDtypeStruct((1, E), jnp.int32),
        ),
    )(x2d, w_router, expert_bias.reshape(1, E))


def _dispatch_body(pos_hbm, x_hbm, xs_hbm, pos_v, rows_v, sem):
    c = lax.axis_index("c")
    s = lax.axis_index("s")
    wid = s * _NC + c
    tbase = wid * TPW
    pltpu.sync_copy(pos_hbm.at[pl.ds(tbase, TPW)], pos_v)
    pltpu.sync_copy(x_hbm.at[pl.ds(tbase, TPW)], rows_v)
    pltpu.async_copy(rows_v, xs_hbm.at[pos_v], sem).wait()


def _dispatch(pos_flat, x2d):
    mesh = plsc.VectorSubcoreMesh(core_axis_name="c", subcore_axis_name="s")
    fn = pl.kernel(
        _dispatch_body,
        out_type=jax.ShapeDtypeStruct((NPAD, D), jnp.float32),
        mesh=mesh,
        scratch_types=[
            pltpu.VMEM((TPW,), jnp.int32),
            pltpu.VMEM((TPW, D), jnp.float32),
            pltpu.SemaphoreType.DMA,
        ],
    )
    return fn(pos_flat, x2d)


def _grouped_body(offs_ref, cnts_ref, xs_ref, w1_ref, w2_ref, w3_ref, out_ref):
    e2 = pl.program_id(0)
    for u in range(EPG):
        e = EPG * e2 + u
        off = pl.multiple_of(offs_ref[e], 8)
        cnt = cnts_ref[e]
        nblk = lax.div(cnt + (BLK - 1), BLK)
        w1 = w1_ref[u]
        w2 = w2_ref[u]
        w3 = w3_ref[u]

        def blk(i, carry):
            base = off + i * BLK
            rows = xs_ref[pl.ds(base, BLK), :]
            h1 = lax.dot_general(rows, w1, (((1,), (1,)), ((), ())),
                                 preferred_element_type=jnp.float32)
            h2 = lax.dot_general(rows, w2, (((1,), (1,)), ((), ())),
                                 preferred_element_type=jnp.float32)
            h = h1 * jax.nn.sigmoid(h1) * h2
            y = lax.dot_general(h, w3, (((1,), (1,)), ((), ())),
                                preferred_element_type=jnp.float32)
            out_ref[pl.ds(base, BLK), :] = y
            return carry

        lax.fori_loop(0, nblk, blk, 0)


def _grouped(offs, cnts, xs, w12, w3):
    return pl.pallas_call(
        _grouped_body,
        grid=(E // EPG,),
        in_specs=[
            pl.BlockSpec(memory_space=pltpu.SMEM),
            pl.BlockSpec(memory_space=pltpu.SMEM),
            pl.BlockSpec((NPAD, D), lambda e: (0, 0)),
            pl.BlockSpec((EPG, RH, D), lambda e: (e, 0, 0)),
            pl.BlockSpec((EPG, RH, D), lambda e: (e, 1, 0)),
            pl.BlockSpec((EPG, D, RH), lambda e: (e, 0, 0)),
        ],
        out_specs=pl.BlockSpec((NPAD, D), lambda e: (0, 0)),
        out_shape=jax.ShapeDtypeStruct((NPAD, D), jnp.float32),
    )(offs, cnts, xs, w12, w12, w3)


def _ungather_body(outs_hbm, pos_hbm, sh_hbm, scb_hbm, final_hbm,
                   pos_v, rows_v, sh_v, sc_v, sem):
    c = lax.axis_index("c")
    s = lax.axis_index("s")
    wid = s * _NC + c
    tbase = wid * TPW
    pltpu.sync_copy(pos_hbm.at[pl.ds(tbase, TPW)], pos_v)
    gather = pltpu.async_copy(outs_hbm.at[pos_v], rows_v, sem)
    pltpu.sync_copy(sh_hbm.at[pl.ds(tbase, TPW)], sh_v)
    pltpu.sync_copy(scb_hbm.at[pl.ds(tbase, TPW)], sc_v)
    gather.wait()

    def tok(t, carry):
        sc16 = sc_v[t, :]
        for j in range(D // 16):
            col = pl.ds(16 * j, 16)
            rows_v[t, col] = sh_v[t, col] + sc16 * rows_v[t, col]
        return carry

    lax.fori_loop(0, TPW, tok, 0)
    pltpu.sync_copy(rows_v, final_hbm.at[pl.ds(tbase, TPW)])


def _ungather_final(out_sorted, pos, shared, scoreb):
    mesh = plsc.VectorSubcoreMesh(core_axis_name="c", subcore_axis_name="s")
    fn = pl.kernel(
        _ungather_body,
        out_type=jax.ShapeDtypeStruct((NT, D), jnp.float32),
        mesh=mesh,
        scratch_types=[
            pltpu.VMEM((TPW,), jnp.int32),
            pltpu.VMEM((TPW, D), jnp.float32),
            pltpu.VMEM((TPW, D), jnp.float32),
            pltpu.VMEM((TPW, 16), jnp.float32),
            pltpu.SemaphoreType.DMA,
        ],
    )
    return fn(out_sorted, pos, shared, scoreb)


def _shared_body(x_ref, w1_ref, w2_ref, w3s_ref, out_ref):
    xb = x_ref[...]
    h1 = lax.dot_general(xb, w1_ref[...], (((1,), (1,)), ((), ())),
                         preferred_element_type=jnp.float32)
    h2 = lax.dot_general(xb, w2_ref[...], (((1,), (1,)), ((), ())),
                         preferred_element_type=jnp.float32)
    h = h1 * jax.nn.sigmoid(h1) * h2
    out_ref[...] = lax.dot_general(h, w3s_ref[...], (((1,), (1,)), ((), ())),
                                   preferred_element_type=jnp.float32)


def _shared(x2d, w1s, w2s, w3s):
    TBLK = 512
    return pl.pallas_call(
        _shared_body,
        grid=(NT // TBLK,),
        in_specs=[
            pl.BlockSpec((TBLK, D), lambda t: (t, 0)),
            pl.BlockSpec((HID, D), lambda t: (0, 0)),
            pl.BlockSpec((HID, D), lambda t: (0, 0)),
            pl.BlockSpec((D, HID), lambda t: (0, 0)),
        ],
        out_specs=pl.BlockSpec((TBLK, D), lambda t: (t, 0)),
        out_shape=jax.ShapeDtypeStruct((NT, D), jnp.float32),
    )(x2d, w1s, w2s, w3s)


def kernel(x, w12, w3, w1s, w2s, w3s, w_router, expert_bias):
    b, s, d = x.shape
    x2d = x.reshape(NT, D)
    scoreb, pos2d, offs2d, cnts2d = _router(x2d, w_router, expert_bias)
    shared = _shared(x2d, w1s, w2s, w3s)
    xs = _dispatch(pos2d.reshape(NT), x2d)
    out_sorted = _grouped(offs2d.reshape(E), cnts2d.reshape(E), xs, w12, w3)
    out = _ungather_final(out_sorted, pos2d.reshape(NT), shared, scoreb)
    return out.reshape(b, s, d)
